# Initial kernel scaffold; baseline (speedup 1.0000x reference)
import jax
import jax.numpy as jnp
from jax.experimental import pallas as pl


def kernel(features, edge_index, W_enc, b_enc, lin1, att_src1, att_dst1, lin2, W_pred, b_pred, W_dec, b_dec):
    raise NotImplementedError("write your pallas kernel here")


# speedup vs baseline: 15.8000x; 15.8000x over previous
import functools
import jax
import jax.numpy as jnp
from jax import lax
from jax.experimental import pallas as pl
from jax.experimental.pallas import tpu as pltpu
from jax.experimental.pallas import tpu_sc as plsc

N = 10000
E = 320000
D = 64
NC = 2
NS = 16
NW = NC * NS
EPW = E // NW
CH = 400
NCHUNK = EPW // CH
SUB = 80
NSUB = CH // SUB
GROUPS = CH // 16
RPN = N // NS
DSUB = 10
DPC = N // DSUB


def _agg_body(x_hbm, src_hbm, dst_hbm, asrc_hbm, adst_hbm, z2_hbm, z1_hbm,
              agg_hbm, den_hbm,
              agg_sh, den_sh, src_v, dst_v, as_v, ad_v, ex_v, rows_v,
              gsem, rsem):
    c = lax.axis_index("c")
    s = lax.axis_index("s")
    wid = c * NS + s

    @pl.when(s < DSUB)
    def _():
        pltpu.sync_copy(z2_hbm.at[pl.ds(s * DPC, DPC)],
                        agg_sh.at[pl.ds(s * DPC, DPC)])
        pltpu.sync_copy(z1_hbm.at[pl.ds(s * DPC, DPC)],
                        den_sh.at[pl.ds(s * DPC, DPC)])

    plsc.subcore_barrier()

    def chunk(i, carry):
        row0 = (wid * NCHUNK + i) * NSUB
        pltpu.sync_copy(src_hbm.at[pl.ds(row0, NSUB)], src_v)
        pltpu.sync_copy(dst_hbm.at[pl.ds(row0, NSUB)], dst_v)
        cps = []
        for k in range(NSUB):
            cps.append(pltpu.async_copy(
                asrc_hbm.at[src_v.at[k]], as_v.at[pl.ds(k * SUB, SUB)], gsem))
        for k in range(NSUB):
            cps.append(pltpu.async_copy(
                adst_hbm.at[dst_v.at[k]], ad_v.at[pl.ds(k * SUB, SUB)], gsem))
        rps = []
        for k in range(NSUB):
            rps.append(pltpu.async_copy(
                x_hbm.at[src_v.at[k]], rows_v.at[pl.ds(k * SUB, SUB)], rsem))
        for cp in cps:
            cp.wait()
        for g in range(GROUPS):
            sl = pl.ds(g * 16, 16)
            t = as_v[sl] + ad_v[sl]
            ex_v[sl] = jnp.exp(jnp.maximum(t, 0.2 * t))
        for rp in rps:
            rp.wait()

        def scale(g, carry2):
            for l in range(16):
                e = g * 16 + l
                b = plsc.load_gather(ex_v, [jnp.full((16,), e, jnp.int32)])
                for q in range(4):
                    qs = pl.ds(q * 16, 16)
                    rows_v[e, qs] = rows_v[e, qs] * b
            return carry2
        lax.fori_loop(0, GROUPS, scale, 0)

        for k in range(NSUB):
            pltpu.sync_copy(rows_v.at[pl.ds(k * SUB, SUB)],
                            agg_sh.at[dst_v.at[k]], add=True)
            pltpu.sync_copy(ex_v.at[pl.ds(k * SUB, SUB)],
                            den_sh.at[dst_v.at[k]], add=True)
        return carry

    lax.fori_loop(0, NCHUNK, chunk, 0)

    plsc.subcore_barrier()

    @pl.when(s < DSUB)
    def _():
        pltpu.sync_copy(agg_sh.at[pl.ds(s * DPC, DPC)],
                        agg_hbm.at[c, pl.ds(s * DPC, DPC)])
        pltpu.sync_copy(den_sh.at[pl.ds(s * DPC, DPC)],
                        den_hbm.at[c, pl.ds(s * DPC, DPC)])


_gat_agg = functools.partial(
    pl.kernel,
    out_type=(jax.ShapeDtypeStruct((NC, N, D), jnp.float32),
              jax.ShapeDtypeStruct((NC, N), jnp.float32)),
    mesh=plsc.VectorSubcoreMesh(core_axis_name="c", subcore_axis_name="s"),
    compiler_params=pltpu.CompilerParams(use_tc_tiling_on_sc=False,
                                         needs_layout_passes=False),
    scratch_types=[
        pltpu.VMEM_SHARED((N, D), jnp.float32),
        pltpu.VMEM_SHARED((N,), jnp.float32),
        pltpu.VMEM((NSUB, SUB), jnp.int32),
        pltpu.VMEM((NSUB, SUB), jnp.int32),
        pltpu.VMEM((CH,), jnp.float32),
        pltpu.VMEM((CH,), jnp.float32),
        pltpu.VMEM((CH,), jnp.float32),
        pltpu.VMEM((CH, D), jnp.float32),
        pltpu.SemaphoreType.DMA,
        pltpu.SemaphoreType.DMA,
    ],
)(_agg_body)


def _pre_body(f_ref, we_ref, be_ref, l1_ref, as_ref, ad_ref,
              x1_ref, a1s_ref, a1d_ref):
    de = jnp.dot(f_ref[...], we_ref[...],
                 preferred_element_type=jnp.float32) + be_ref[...]
    x1 = jnp.dot(de, l1_ref[...], preferred_element_type=jnp.float32)
    x1_ref[...] = x1
    a1s_ref[...] = jnp.sum(x1 * as_ref[...], axis=-1, keepdims=True)
    a1d_ref[...] = jnp.sum(x1 * ad_ref[...], axis=-1, keepdims=True)


_pre = pl.pallas_call(
    _pre_body,
    out_shape=(jax.ShapeDtypeStruct((N, D), jnp.float32),
               jax.ShapeDtypeStruct((N, 1), jnp.float32),
               jax.ShapeDtypeStruct((N, 1), jnp.float32)),
)


def _elu(x):
    return jnp.where(x > 0, x, jnp.exp(jnp.minimum(x, 0.0)) - 1.0)


def _mid_body(agg_ref, den_ref, l2_ref, l2t_ref, wp_ref, bp_ref,
              h2_ref, lsm_ref, x3_ref):
    a = agg_ref[...]
    d = den_ref[...]
    r = 1.0 / (d[0] + d[1] + 1e-16)
    h1 = _elu((a[0] + a[1]) * r)
    h2 = jnp.dot(h1, l2_ref[...], preferred_element_type=jnp.float32)
    h2_ref[...] = h2
    pred = jnp.dot(h2, wp_ref[...],
                   preferred_element_type=jnp.float32) + bp_ref[...]
    m = jnp.max(pred, axis=-1, keepdims=True)
    lse = jnp.log(jnp.sum(jnp.exp(pred - m), axis=-1, keepdims=True)) + m
    lsm_ref[...] = pred - lse
    x3_ref[...] = jnp.dot(h2, l2t_ref[...], preferred_element_type=jnp.float32)


_mid = pl.pallas_call(
    _mid_body,
    out_shape=(jax.ShapeDtypeStruct((N, 32), jnp.float32),
               jax.ShapeDtypeStruct((N, 16), jnp.float32),
               jax.ShapeDtypeStruct((N, D), jnp.float32)),
)


def _post_body(agg_ref, den_ref, l1t_ref, wd_ref, bd_ref, out_ref):
    a = agg_ref[...]
    d = den_ref[...]
    r = 1.0 / (d[0] + d[1] + 1e-16)
    h3 = _elu((a[0] + a[1]) * r)
    h4 = jnp.dot(h3, l1t_ref[...], preferred_element_type=jnp.float32)
    out_ref[...] = jnp.dot(_elu(h4), wd_ref[...],
                           preferred_element_type=jnp.float32) + bd_ref[...]


_post = pl.pallas_call(
    _post_body,
    out_shape=jax.ShapeDtypeStruct((N, 128), jnp.float32),
)


def kernel(features, edge_index, W_enc, b_enc, lin1, att_src1, att_dst1,
           lin2, W_pred, b_pred, W_dec, b_dec):
    src = edge_index[0].reshape(E // SUB, SUB)
    dst = edge_index[1].reshape(E // SUB, SUB)
    x1, a1s, a1d = _pre(features, W_enc, b_enc.reshape(1, -1), lin1,
                        att_src1.reshape(1, -1), att_dst1.reshape(1, -1))
    asrc = a1s.reshape(N)
    adst = a1d.reshape(N)
    z2 = jnp.zeros((N, D), jnp.float32)
    z1 = jnp.zeros((N,), jnp.float32)
    agg1, den = _gat_agg(x1, src, dst, asrc, adst, z2, z1)
    den3 = den.reshape(NC, N, 1)
    h2, lsm, x3 = _mid(agg1, den3, lin2, lin2.T, W_pred, b_pred.reshape(1, -1))
    agg3, _ = _gat_agg(x3, src, dst, asrc, adst, z2, z1)
    out = _post(agg3, den3, lin1.T, W_dec, b_dec.reshape(1, -1))
    return (h2, out, lsm)


# speedup vs baseline: 33.1893x; 2.1006x over previous
import functools
import jax
import jax.numpy as jnp
from jax import lax
from jax.experimental import pallas as pl
from jax.experimental.pallas import tpu as pltpu
from jax.experimental.pallas import tpu_sc as plsc

N = 10000
E = 320000
D = 64
NC = 2
NS = 16
NW = NC * NS
EPW = E // NW
CH = 400
NCHUNK = EPW // CH
SUB = 80
NSUB = CH // SUB
GROUPS = CH // 16
RPN = N // NS
DSUB = 10
DPC = N // DSUB


IPW = EPW // SUB


def _agg_body(x_hbm, src_hbm, dst_hbm, asrc_hbm, adst_hbm, z2_hbm, z1_hbm,
              agg_hbm, den_hbm,
              agg_sh, den_sh, srcall, dstall, as_v, ad_v, ex_v, rows_v,
              gsem, rsem, ssem):
    c = lax.axis_index("c")
    s = lax.axis_index("s")
    wid = c * NS + s

    @pl.when(s < DSUB)
    def _():
        pltpu.sync_copy(z2_hbm.at[pl.ds(s * DPC, DPC)],
                        agg_sh.at[pl.ds(s * DPC, DPC)])
        pltpu.sync_copy(z1_hbm.at[pl.ds(s * DPC, DPC)],
                        den_sh.at[pl.ds(s * DPC, DPC)])

    pltpu.sync_copy(src_hbm.at[pl.ds(wid * IPW, IPW)], srcall)
    pltpu.sync_copy(dst_hbm.at[pl.ds(wid * IPW, IPW)], dstall)

    plsc.subcore_barrier()

    def issue_gathers(j, off):
        for k in range(NSUB):
            r = j * NSUB + k
            o = pl.ds(off + k * SUB, SUB)
            pltpu.async_copy(asrc_hbm.at[srcall.at[r]], as_v.at[o], gsem)
            pltpu.async_copy(adst_hbm.at[dstall.at[r]], ad_v.at[o], gsem)
            pltpu.async_copy(x_hbm.at[srcall.at[r]],
                             rows_v.at[pl.ds(off + k * SUB, SUB)], rsem)

    issue_gathers(0, 0)

    def chunk(i, carry):
        off = lax.rem(i, 2) * CH
        ooff = CH - off

        pltpu.make_async_copy(
            asrc_hbm.at[pl.ds(0, CH)], as_v.at[pl.ds(off, CH)], gsem).wait()
        pltpu.make_async_copy(
            adst_hbm.at[pl.ds(0, CH)], ad_v.at[pl.ds(off, CH)], gsem).wait()
        pltpu.make_async_copy(
            x_hbm.at[pl.ds(0, CH)], rows_v.at[pl.ds(off, CH)], rsem).wait()

        @pl.when(i > 0)
        def _():
            pltpu.make_async_copy(
                x_hbm.at[pl.ds(0, CH)], rows_v.at[pl.ds(ooff, CH)],
                ssem).wait()
            pltpu.make_async_copy(
                asrc_hbm.at[pl.ds(0, CH)], ex_v.at[pl.ds(ooff, CH)],
                ssem).wait()

        @pl.when(i < NCHUNK - 1)
        def _():
            issue_gathers(i + 1, ooff)

        for g in range(GROUPS):
            sl = pl.ds(off + g * 16, 16)
            t = as_v[sl] + ad_v[sl]
            ex_v[sl] = jnp.exp(jnp.maximum(t, 0.2 * t))

        def scale(g, carry2):
            for l in range(16):
                e = off + g * 16 + l
                b = plsc.load_gather(ex_v, [jnp.full((16,), e, jnp.int32)])
                for q in range(4):
                    qs = pl.ds(q * 16, 16)
                    rows_v[e, qs] = rows_v[e, qs] * b
            return carry2
        lax.fori_loop(0, GROUPS, scale, 0)

        for k in range(NSUB):
            r = i * NSUB + k
            pltpu.async_copy(rows_v.at[pl.ds(off + k * SUB, SUB)],
                             agg_sh.at[dstall.at[r]], ssem, add=True)
            pltpu.async_copy(ex_v.at[pl.ds(off + k * SUB, SUB)],
                             den_sh.at[dstall.at[r]], ssem, add=True)
        return carry

    lax.fori_loop(0, NCHUNK, chunk, 0)

    loff = lax.rem(NCHUNK - 1, 2) * CH
    pltpu.make_async_copy(
        x_hbm.at[pl.ds(0, CH)], rows_v.at[pl.ds(loff, CH)], ssem).wait()
    pltpu.make_async_copy(
        asrc_hbm.at[pl.ds(0, CH)], ex_v.at[pl.ds(loff, CH)], ssem).wait()

    plsc.subcore_barrier()

    @pl.when(s < DSUB)
    def _():
        pltpu.sync_copy(agg_sh.at[pl.ds(s * DPC, DPC)],
                        agg_hbm.at[c, pl.ds(s * DPC, DPC)])
        pltpu.sync_copy(den_sh.at[pl.ds(s * DPC, DPC)],
                        den_hbm.at[c, pl.ds(s * DPC, DPC)])


_gat_agg = functools.partial(
    pl.kernel,
    out_type=(jax.ShapeDtypeStruct((NC, N, D), jnp.float32),
              jax.ShapeDtypeStruct((NC, N), jnp.float32)),
    mesh=plsc.VectorSubcoreMesh(core_axis_name="c", subcore_axis_name="s"),
    compiler_params=pltpu.CompilerParams(use_tc_tiling_on_sc=False,
                                         needs_layout_passes=False),
    scratch_types=[
        pltpu.VMEM_SHARED((N, D), jnp.float32),
        pltpu.VMEM_SHARED((N,), jnp.float32),
        pltpu.VMEM((IPW, SUB), jnp.int32),
        pltpu.VMEM((IPW, SUB), jnp.int32),
        pltpu.VMEM((2 * CH,), jnp.float32),
        pltpu.VMEM((2 * CH,), jnp.float32),
        pltpu.VMEM((2 * CH,), jnp.float32),
        pltpu.VMEM((2 * CH, D), jnp.float32),
        pltpu.SemaphoreType.DMA,
        pltpu.SemaphoreType.DMA,
        pltpu.SemaphoreType.DMA,
    ],
)(_agg_body)


def _pre_body(f_ref, we_ref, be_ref, l1_ref, as_ref, ad_ref,
              x1_ref, a1s_ref, a1d_ref):
    de = jnp.dot(f_ref[...], we_ref[...],
                 preferred_element_type=jnp.float32) + be_ref[...]
    x1 = jnp.dot(de, l1_ref[...], preferred_element_type=jnp.float32)
    x1_ref[...] = x1
    a1s_ref[...] = jnp.sum(x1 * as_ref[...], axis=-1, keepdims=True)
    a1d_ref[...] = jnp.sum(x1 * ad_ref[...], axis=-1, keepdims=True)


_pre = pl.pallas_call(
    _pre_body,
    out_shape=(jax.ShapeDtypeStruct((N, D), jnp.float32),
               jax.ShapeDtypeStruct((N, 1), jnp.float32),
               jax.ShapeDtypeStruct((N, 1), jnp.float32)),
)


def _elu(x):
    return jnp.where(x > 0, x, jnp.exp(jnp.minimum(x, 0.0)) - 1.0)


def _mid_body(agg_ref, den_ref, l2_ref, l2t_ref, wp_ref, bp_ref,
              h2_ref, lsm_ref, x3_ref):
    a = agg_ref[...]
    d = den_ref[...]
    r = 1.0 / (d[0] + d[1] + 1e-16)
    h1 = _elu((a[0] + a[1]) * r)
    h2 = jnp.dot(h1, l2_ref[...], preferred_element_type=jnp.float32)
    h2_ref[...] = h2
    pred = jnp.dot(h2, wp_ref[...],
                   preferred_element_type=jnp.float32) + bp_ref[...]
    m = jnp.max(pred, axis=-1, keepdims=True)
    lse = jnp.log(jnp.sum(jnp.exp(pred - m), axis=-1, keepdims=True)) + m
    lsm_ref[...] = pred - lse
    x3_ref[...] = jnp.dot(h2, l2t_ref[...], preferred_element_type=jnp.float32)


_mid = pl.pallas_call(
    _mid_body,
    out_shape=(jax.ShapeDtypeStruct((N, 32), jnp.float32),
               jax.ShapeDtypeStruct((N, 16), jnp.float32),
               jax.ShapeDtypeStruct((N, D), jnp.float32)),
)


def _post_body(agg_ref, den_ref, l1t_ref, wd_ref, bd_ref, out_ref):
    a = agg_ref[...]
    d = den_ref[...]
    r = 1.0 / (d[0] + d[1] + 1e-16)
    h3 = _elu((a[0] + a[1]) * r)
    h4 = jnp.dot(h3, l1t_ref[...], preferred_element_type=jnp.float32)
    out_ref[...] = jnp.dot(_elu(h4), wd_ref[...],
                           preferred_element_type=jnp.float32) + bd_ref[...]


_post = pl.pallas_call(
    _post_body,
    out_shape=jax.ShapeDtypeStruct((N, 128), jnp.float32),
)


def kernel(features, edge_index, W_enc, b_enc, lin1, att_src1, att_dst1,
           lin2, W_pred, b_pred, W_dec, b_dec):
    src = edge_index[0].reshape(E // SUB, SUB)
    dst = edge_index[1].reshape(E // SUB, SUB)
    x1, a1s, a1d = _pre(features, W_enc, b_enc.reshape(1, -1), lin1,
                        att_src1.reshape(1, -1), att_dst1.reshape(1, -1))
    asrc = a1s.reshape(N)
    adst = a1d.reshape(N)
    z2 = jnp.zeros((N, D), jnp.float32)
    z1 = jnp.zeros((N,), jnp.float32)
    agg1, den = _gat_agg(x1, src, dst, asrc, adst, z2, z1)
    den3 = den.reshape(NC, N, 1)
    h2, lsm, x3 = _mid(agg1, den3, lin2, lin2.T, W_pred, b_pred.reshape(1, -1))
    agg3, _ = _gat_agg(x3, src, dst, asrc, adst, z2, z1)
    out = _post(agg3, den3, lin1.T, W_dec, b_dec.reshape(1, -1))
    return (h2, out, lsm)


# speedup vs baseline: 33.7616x; 1.0172x over previous
import functools
import jax
import jax.numpy as jnp
from jax import lax
from jax.experimental import pallas as pl
from jax.experimental.pallas import tpu as pltpu
from jax.experimental.pallas import tpu_sc as plsc

N = 10000
E = 320000
D = 64
NC = 2
NS = 16
NW = NC * NS
EPW = E // NW
CH = 400
NCHUNK = EPW // CH
SUB = 80
NSUB = CH // SUB
GROUPS = CH // 16
RPN = N // NS
DSUB = 10
DPC = N // DSUB


IPW = EPW // SUB


def _make_agg(first):
    def body(*refs):
        if first:
            (x_hbm, src_hbm, dst_hbm, asrc_hbm, adst_hbm, z2_hbm, z1_hbm,
             agg_hbm, den_hbm, ex_hbm,
             agg_sh, den_sh, srcall, dstall, as_v, ad_v, ex_v, rows_v,
             gsem, rsem, ssem, wsem) = refs
        else:
            (x_hbm, src_hbm, dst_hbm, exin_hbm, z2_hbm,
             agg_hbm,
             agg_sh, srcall, dstall, ex_v, rows_v,
             gsem, rsem, ssem) = refs

        c = lax.axis_index("c")
        s = lax.axis_index("s")
        wid = c * NS + s

        @pl.when(s < DSUB)
        def _():
            pltpu.sync_copy(z2_hbm.at[pl.ds(s * DPC, DPC)],
                            agg_sh.at[pl.ds(s * DPC, DPC)])
            if first:
                pltpu.sync_copy(z1_hbm.at[pl.ds(s * DPC, DPC)],
                                den_sh.at[pl.ds(s * DPC, DPC)])

        pltpu.sync_copy(src_hbm.at[pl.ds(wid * IPW, IPW)], srcall)
        pltpu.sync_copy(dst_hbm.at[pl.ds(wid * IPW, IPW)], dstall)

        plsc.subcore_barrier()

        def issue_gathers(j, off, sb):
            for k in range(NSUB):
                r = j * NSUB + k
                pltpu.async_copy(x_hbm.at[srcall.at[r]],
                                 rows_v.at[pl.ds(off + k * SUB, SUB)], rsem)
            if first:
                for k in range(NSUB):
                    r = j * NSUB + k
                    o = pl.ds(off + k * SUB, SUB)
                    pltpu.async_copy(asrc_hbm.at[srcall.at[r]],
                                     as_v.at[o], gsem)
                    pltpu.async_copy(adst_hbm.at[dstall.at[r]],
                                     ad_v.at[o], gsem)
            else:
                pltpu.async_copy(exin_hbm.at[pl.ds(wid * EPW + j * CH, CH)],
                                 ex_v.at[pl.ds(off, CH)], gsem)

        issue_gathers(0, 0, 0)

        def chunk(i, carry):
            sb = lax.rem(i, 2)
            off = sb * CH
            ooff = CH - off
            osb = 1 - sb

            pltpu.make_async_copy(
                x_hbm.at[pl.ds(0, CH)], rows_v.at[pl.ds(off, CH)],
                rsem).wait()
            if first:
                pltpu.make_async_copy(
                    asrc_hbm.at[pl.ds(0, CH)], as_v.at[pl.ds(off, CH)],
                    gsem).wait()
                pltpu.make_async_copy(
                    asrc_hbm.at[pl.ds(0, CH)], ad_v.at[pl.ds(off, CH)],
                    gsem).wait()
            else:
                pltpu.make_async_copy(
                    exin_hbm.at[pl.ds(0, CH)], ex_v.at[pl.ds(off, CH)],
                    gsem).wait()

            @pl.when(i > 0)
            def _():
                pltpu.make_async_copy(
                    x_hbm.at[pl.ds(0, CH)], rows_v.at[pl.ds(ooff, CH)],
                    ssem).wait()
                if first:
                    pltpu.make_async_copy(
                        asrc_hbm.at[pl.ds(0, CH)], ex_v.at[pl.ds(ooff, CH)],
                        ssem).wait()
                    pltpu.make_async_copy(
                        asrc_hbm.at[pl.ds(0, CH)], ex_v.at[pl.ds(ooff, CH)],
                        wsem).wait()

            @pl.when(i < NCHUNK - 1)
            def _():
                issue_gathers(i + 1, ooff, osb)

            if first:
                for g in range(GROUPS):
                    sl = pl.ds(off + g * 16, 16)
                    t = as_v[sl] + ad_v[sl]
                    ex_v[sl] = jnp.exp(jnp.maximum(t, 0.2 * t))

            def scale(g, carry2):
                for l in range(16):
                    e = off + g * 16 + l
                    b = plsc.load_gather(
                        ex_v, [jnp.full((16,), e, jnp.int32)])
                    for q in range(4):
                        qs = pl.ds(q * 16, 16)
                        rows_v[e, qs] = rows_v[e, qs] * b
                return carry2
            lax.fori_loop(0, GROUPS, scale, 0)

            for k in range(NSUB):
                r = i * NSUB + k
                pltpu.async_copy(rows_v.at[pl.ds(off + k * SUB, SUB)],
                                 agg_sh.at[dstall.at[r]], ssem, add=True)
                if first:
                    pltpu.async_copy(ex_v.at[pl.ds(off + k * SUB, SUB)],
                                     den_sh.at[dstall.at[r]], ssem, add=True)
            if first:
                pltpu.async_copy(ex_v.at[pl.ds(off, CH)],
                                 ex_hbm.at[pl.ds(wid * EPW + i * CH, CH)],
                                 wsem)
            return carry

        lax.fori_loop(0, NCHUNK, chunk, 0)

        loff = ((NCHUNK - 1) % 2) * CH
        pltpu.make_async_copy(
            x_hbm.at[pl.ds(0, CH)], rows_v.at[pl.ds(loff, CH)], ssem).wait()
        if first:
            pltpu.make_async_copy(
                asrc_hbm.at[pl.ds(0, CH)], ex_v.at[pl.ds(loff, CH)],
                ssem).wait()
            pltpu.make_async_copy(
                asrc_hbm.at[pl.ds(0, CH)], ex_v.at[pl.ds(loff, CH)],
                wsem).wait()

        plsc.subcore_barrier()

        @pl.when(s < DSUB)
        def _():
            pltpu.sync_copy(agg_sh.at[pl.ds(s * DPC, DPC)],
                            agg_hbm.at[c, pl.ds(s * DPC, DPC)])
            if first:
                pltpu.sync_copy(den_sh.at[pl.ds(s * DPC, DPC)],
                                den_hbm.at[c, pl.ds(s * DPC, DPC)])

    if first:
        out_type = (jax.ShapeDtypeStruct((NC, N, D), jnp.float32),
                    jax.ShapeDtypeStruct((NC, N), jnp.float32),
                    jax.ShapeDtypeStruct((E,), jnp.float32))
        scratch = [
            pltpu.VMEM_SHARED((N, D), jnp.float32),
            pltpu.VMEM_SHARED((N,), jnp.float32),
            pltpu.VMEM((IPW, SUB), jnp.int32),
            pltpu.VMEM((IPW, SUB), jnp.int32),
            pltpu.VMEM((2 * CH,), jnp.float32),
            pltpu.VMEM((2 * CH,), jnp.float32),
            pltpu.VMEM((2 * CH,), jnp.float32),
            pltpu.VMEM((2 * CH, D), jnp.float32),
            pltpu.SemaphoreType.DMA,
            pltpu.SemaphoreType.DMA,
            pltpu.SemaphoreType.DMA,
            pltpu.SemaphoreType.DMA,
        ]
    else:
        out_type = jax.ShapeDtypeStruct((NC, N, D), jnp.float32)
        scratch = [
            pltpu.VMEM_SHARED((N, D), jnp.float32),
            pltpu.VMEM((IPW, SUB), jnp.int32),
            pltpu.VMEM((IPW, SUB), jnp.int32),
            pltpu.VMEM((2 * CH,), jnp.float32),
            pltpu.VMEM((2 * CH, D), jnp.float32),
            pltpu.SemaphoreType.DMA,
            pltpu.SemaphoreType.DMA,
            pltpu.SemaphoreType.DMA,
        ]
    return pl.kernel(
        body,
        out_type=out_type,
        mesh=plsc.VectorSubcoreMesh(core_axis_name="c", subcore_axis_name="s"),
        compiler_params=pltpu.CompilerParams(use_tc_tiling_on_sc=False,
                                             needs_layout_passes=False),
        scratch_types=scratch,
    )


_gat_agg1 = _make_agg(True)
_gat_agg2 = _make_agg(False)


def _pre_body(f_ref, we_ref, be_ref, l1_ref, as_ref, ad_ref,
              x1_ref, a1s_ref, a1d_ref):
    de = jnp.dot(f_ref[...], we_ref[...],
                 preferred_element_type=jnp.float32) + be_ref[...]
    x1 = jnp.dot(de, l1_ref[...], preferred_element_type=jnp.float32)
    x1_ref[...] = x1
    a1s_ref[...] = jnp.sum(x1 * as_ref[...], axis=-1, keepdims=True)
    a1d_ref[...] = jnp.sum(x1 * ad_ref[...], axis=-1, keepdims=True)


_pre = pl.pallas_call(
    _pre_body,
    out_shape=(jax.ShapeDtypeStruct((N, D), jnp.float32),
               jax.ShapeDtypeStruct((N, 1), jnp.float32),
               jax.ShapeDtypeStruct((N, 1), jnp.float32)),
)


def _elu(x):
    return jnp.where(x > 0, x, jnp.exp(jnp.minimum(x, 0.0)) - 1.0)


def _mid_body(agg_ref, den_ref, l2_ref, l2t_ref, wp_ref, bp_ref,
              h2_ref, lsm_ref, x3_ref):
    a = agg_ref[...]
    d = den_ref[...]
    r = 1.0 / (d[0] + d[1] + 1e-16)
    h1 = _elu((a[0] + a[1]) * r)
    h2 = jnp.dot(h1, l2_ref[...], preferred_element_type=jnp.float32)
    h2_ref[...] = h2
    pred = jnp.dot(h2, wp_ref[...],
                   preferred_element_type=jnp.float32) + bp_ref[...]
    m = jnp.max(pred, axis=-1, keepdims=True)
    lse = jnp.log(jnp.sum(jnp.exp(pred - m), axis=-1, keepdims=True)) + m
    lsm_ref[...] = pred - lse
    x3_ref[...] = jnp.dot(h2, l2t_ref[...], preferred_element_type=jnp.float32)


_mid = pl.pallas_call(
    _mid_body,
    out_shape=(jax.ShapeDtypeStruct((N, 32), jnp.float32),
               jax.ShapeDtypeStruct((N, 16), jnp.float32),
               jax.ShapeDtypeStruct((N, D), jnp.float32)),
)


def _post_body(agg_ref, den_ref, l1t_ref, wd_ref, bd_ref, out_ref):
    a = agg_ref[...]
    d = den_ref[...]
    r = 1.0 / (d[0] + d[1] + 1e-16)
    h3 = _elu((a[0] + a[1]) * r)
    h4 = jnp.dot(h3, l1t_ref[...], preferred_element_type=jnp.float32)
    out_ref[...] = jnp.dot(_elu(h4), wd_ref[...],
                           preferred_element_type=jnp.float32) + bd_ref[...]


_post = pl.pallas_call(
    _post_body,
    out_shape=jax.ShapeDtypeStruct((N, 128), jnp.float32),
)


def kernel(features, edge_index, W_enc, b_enc, lin1, att_src1, att_dst1,
           lin2, W_pred, b_pred, W_dec, b_dec):
    src = edge_index[0].reshape(E // SUB, SUB)
    dst = edge_index[1].reshape(E // SUB, SUB)
    x1, a1s, a1d = _pre(features, W_enc, b_enc.reshape(1, -1), lin1,
                        att_src1.reshape(1, -1), att_dst1.reshape(1, -1))
    asrc = a1s.reshape(N)
    adst = a1d.reshape(N)
    z2 = jnp.zeros((N, D), jnp.float32)
    z1 = jnp.zeros((N,), jnp.float32)
    agg1, den, ex = _gat_agg1(x1, src, dst, asrc, adst, z2, z1)
    den3 = den.reshape(NC, N, 1)
    h2, lsm, x3 = _mid(agg1, den3, lin2, lin2.T, W_pred, b_pred.reshape(1, -1))
    agg3 = _gat_agg2(x3, src, dst, ex, z2)
    out = _post(agg3, den3, lin1.T, W_dec, b_dec.reshape(1, -1))
    return (h2, out, lsm)
